# baseline (speedup 1.0000x reference)
import functools

import jax
import jax.numpy as jnp
from jax import lax
from jax.experimental import pallas as pl
from jax.experimental.pallas import tpu as pltpu
from jax.experimental.pallas import tpu_sc as plsc

HOP = 5
N = 10000
NPAD = 10240
H = 128
HH = 64
E = 320000
CH = 128
NCH = 162
NPAIR = NCH // 3
EPAD = 16 * NCH * CH
SLAB = NPAD // 16
DUMMY = 10016


def _lrelu(v):
    return jnp.where(v > 0, v, 0.2 * v)


def _agg_kernel(hap0, hap1, hpa0, hpa1, s_ap, t_ap, s_pa, t_pa, scal_ap, scal_pa,
                accap0, accap1, seg_ap, accpa0, accpa1, seg_pa,
                x1buf, h1buf, sidx, tidx, rows0, rows1, rows2, wbuf,
                accsh, segsh, gsem0, gsem1, gsem2, ssem0, ssem1, ssem2):
    sid = lax.axis_index("s")
    cid = lax.axis_index("c")
    zero16 = jnp.zeros((16,), jnp.float32)
    rows_bufs = (rows0, rows1, rows2)
    gsems = (gsem0, gsem1, gsem2)
    ssems = (ssem0, ssem1, ssem2)

    def _run_type(h0, h1, s2d, t2d, scal, acc0_out, acc1_out, seg_out):
        pltpu.sync_copy(scal.at[0], x1buf)
        pltpu.sync_copy(scal.at[1], h1buf)
        tile_row0 = sid * NCH
        pltpu.sync_copy(s2d.at[pl.ds(tile_row0, NCH)], sidx)
        pltpu.sync_copy(t2d.at[pl.ds(tile_row0, NCH)], tidx)

        for p in range(2):
            hs = h0 if p == 0 else h1
            acc_out = acc0_out if p == 0 else acc1_out

            def _fire_gather(c, b):
                return pltpu.async_copy(hs.at[tidx.at[c]], rows_bufs[b],
                                        gsems[b])

            def _wait_gather(c, b):
                pltpu.make_async_copy(hs.at[tidx.at[c]], rows_bufs[b],
                                      gsems[b]).wait()

            def _fire_scatter(c, b):
                pltpu.async_copy(rows_bufs[b], accsh.at[sidx.at[c]],
                                 ssems[b], add=True)
                if p == 0:
                    pltpu.async_copy(wbuf.at[b], segsh.at[sidx.at[c]],
                                     ssems[b], add=True)

            def _wait_scatter(c, b):
                pltpu.make_async_copy(rows_bufs[b], accsh.at[sidx.at[c]],
                                      ssems[b]).wait()
                if p == 0:
                    pltpu.make_async_copy(wbuf.at[b], segsh.at[sidx.at[c]],
                                          ssems[b]).wait()

            def _zrow(i, cc):
                for q in range(HH // 16):
                    rows0[i, pl.ds(q * 16, 16)] = zero16
                return cc
            lax.fori_loop(0, CH, _zrow, 0)
            for r in range(SLAB // CH):
                pltpu.sync_copy(rows0,
                                accsh.at[pl.ds(sid * SLAB + r * CH, CH)])
            if p == 0:
                for q in range(8):
                    wbuf[0, pl.ds(q * 16, 16)] = zero16
                for r in range(SLAB // 128):
                    pltpu.sync_copy(wbuf.at[0],
                                    segsh.at[pl.ds(sid * SLAB + r * 128, 128)])
            plsc.subcore_barrier()

            _fire_gather(0, 0)

            def _pair(g, carry):
                for db in range(3):
                    c = g * 3 + db
                    b = db
                    nb = (db + 1) % 3
                    if db < 2:
                        @pl.when(c >= 2)
                        def _():
                            _wait_scatter(c - 2, nb)
                        _fire_gather(c + 1, nb)
                    else:
                        @pl.when(g < NPAIR - 1)
                        def _():
                            _wait_scatter(c - 2, nb)
                            _fire_gather(c + 1, nb)
                    _wait_gather(c, b)
                    rows = rows_bufs[b]
                    def _wg(gg, cc):
                        col = gg * 16
                        sv = sidx[c, pl.ds(col, 16)]
                        tv = tidx[c, pl.ds(col, 16)]
                        xv = plsc.load_gather(x1buf, [sv])
                        hv = plsc.load_gather(h1buf, [tv])
                        wbuf[b, pl.ds(col, 16)] = (
                            jnp.exp(_lrelu(xv + hv)))
                        return cc
                    lax.fori_loop(0, 8, _wg, 0)
                    wvs = [wbuf[b, pl.ds(g * 16, 16)] for g in range(8)]
                    iotas = [lax.iota(jnp.int32, 16) + g * 16
                             for g in range(8)]

                    def _scale(col, cc):
                        cvec = jnp.full((16,), col, jnp.int32)
                        for g in range(8):
                            v = plsc.load_gather(rows, [iotas[g], cvec])
                            plsc.store_scatter(rows, [iotas[g], cvec],
                                               v * wvs[g])
                        return cc
                    lax.fori_loop(0, HH, _scale, 0)
                    _fire_scatter(c, b)
                return carry

            lax.fori_loop(0, NPAIR, _pair, 0)
            for c, b in ((NCH - 3, 0), (NCH - 2, 1), (NCH - 1, 2)):
                _wait_scatter(c, b)
            plsc.subcore_barrier()
            pltpu.sync_copy(accsh.at[pl.ds(sid * SLAB, SLAB)],
                            acc_out.at[pl.ds(sid * SLAB, SLAB)])
            if p == 0:
                pltpu.sync_copy(segsh.at[pl.ds(sid * SLAB, SLAB)],
                                seg_out.at[pl.ds(sid * SLAB, SLAB)])

    @pl.when(cid == 0)
    def _():
        _run_type(hap0, hap1, s_ap, t_ap, scal_ap, accap0, accap1, seg_ap)

    @pl.when(cid == 1)
    def _():
        _run_type(hpa0, hpa1, s_pa, t_pa, scal_pa, accpa0, accpa1, seg_pa)


_agg = functools.partial(
    pl.kernel,
    mesh=plsc.VectorSubcoreMesh(core_axis_name="c", subcore_axis_name="s"),
    compiler_params=pltpu.CompilerParams(
        needs_layout_passes=False, use_tc_tiling_on_sc=False),
    out_type=[
        jax.ShapeDtypeStruct((NPAD, HH), jnp.float32),
        jax.ShapeDtypeStruct((NPAD, HH), jnp.float32),
        jax.ShapeDtypeStruct((NPAD,), jnp.float32),
        jax.ShapeDtypeStruct((NPAD, HH), jnp.float32),
        jax.ShapeDtypeStruct((NPAD, HH), jnp.float32),
        jax.ShapeDtypeStruct((NPAD,), jnp.float32),
    ],
    scratch_types=[
        pltpu.VMEM((NPAD,), jnp.float32),
        pltpu.VMEM((NPAD,), jnp.float32),
        pltpu.VMEM((NCH, 128), jnp.int32),
        pltpu.VMEM((NCH, 128), jnp.int32),
        pltpu.VMEM((CH, HH), jnp.float32),
        pltpu.VMEM((CH, HH), jnp.float32),
        pltpu.VMEM((CH, HH), jnp.float32),
        pltpu.VMEM((3, 128), jnp.float32),
        pltpu.VMEM_SHARED((NPAD, HH), jnp.float32),
        pltpu.VMEM_SHARED((NPAD,), jnp.float32),
        pltpu.SemaphoreType.DMA,
        pltpu.SemaphoreType.DMA,
        pltpu.SemaphoreType.DMA,
        pltpu.SemaphoreType.DMA,
        pltpu.SemaphoreType.DMA,
        pltpu.SemaphoreType.DMA,
    ],
)(_agg_kernel)


def _proj_body(h_ref, w_ref, b_ref, o_ref):
    o_ref[...] = h_ref[...] @ w_ref[...] + b_ref[...]


def _pad_edges(ei):
    s = jnp.concatenate([ei[0], jnp.full((EPAD - E,), DUMMY, jnp.int32)])
    t = jnp.concatenate([ei[1], jnp.zeros((EPAD - E,), jnp.int32)])
    return s.reshape(EPAD // 128, 128), t.reshape(EPAD // 128, 128)


def _pad_vec(v):
    return jnp.concatenate([v, jnp.zeros((NPAD - N,), jnp.float32)])


def kernel(x_author, x_paper, edge_index_ap, edge_index_pa, W1_author, b1_author, W1_paper, b1_paper, attn1, attn2, W_out, b_out):
    xa = jax.nn.relu(x_author @ W1_author + b1_author)
    xp = jax.nn.relu(x_paper @ W1_paper + b1_paper)
    s_ap, t_ap = _pad_edges(edge_index_ap)
    s_pa, t_pa = _pad_edges(edge_index_pa)

    A1_ap = attn1[:, 0, :].T
    A2_ap = attn2[:, 0, :].T
    A1_pa = attn1[:, 1, :].T
    A2_pa = attn2[:, 1, :].T
    x1_ap_all = xa @ A1_ap
    x1_pa_all = xp @ A1_pa
    w2_ap_all = jnp.exp(_lrelu(x1_ap_all + xa @ A2_ap))
    w2_pa_all = jnp.exp(_lrelu(x1_pa_all + xp @ A2_pa))

    ha, hp = xa, xp
    for i in range(HOP):
        h1_ap = _pad_vec(hp @ attn2[i, 0])
        h1_pa = _pad_vec(ha @ attn2[i, 1])
        scal_ap = jnp.stack([_pad_vec(x1_ap_all[:, i]), h1_ap])
        scal_pa = jnp.stack([_pad_vec(x1_pa_all[:, i]), h1_pa])
        acc_ap0, acc_ap1, seg_ap, acc_pa0, acc_pa1, seg_pa = _agg(
            hp[:, :HH], hp[:, HH:], ha[:, :HH], ha[:, HH:],
            s_ap, t_ap, s_pa, t_pa, scal_ap, scal_pa)
        acc_a = jnp.concatenate([acc_ap0[:N], acc_ap1[:N]], axis=1)
        acc_p = jnp.concatenate([acc_pa0[:N], acc_pa1[:N]], axis=1)
        w2a = w2_ap_all[:, i]
        w2p = w2_pa_all[:, i]
        hn_a = (acc_a + w2a[:, None] * xa) / (seg_ap[:N] + w2a)[:, None]
        hn_p = (acc_p + w2p[:, None] * xp) / (seg_pa[:N] + w2p)[:, None]
        ha = jax.nn.elu(hn_a)
        hp = jax.nn.elu(hn_p)

    BR = 400
    OUT = W_out.shape[1]
    return pl.pallas_call(
        _proj_body,
        grid=(N // BR,),
        in_specs=[
            pl.BlockSpec((BR, H), lambda i: (i, 0)),
            pl.BlockSpec((H, OUT), lambda i: (0, 0)),
            pl.BlockSpec((OUT,), lambda i: (0,)),
        ],
        out_specs=pl.BlockSpec((BR, OUT), lambda i: (i, 0)),
        out_shape=jax.ShapeDtypeStruct((N, OUT), jnp.float32),
    )(ha, W_out, b_out)


# speedup vs baseline: 3.4441x; 3.4441x over previous
import functools

import jax
import jax.numpy as jnp
from jax import lax
from jax.experimental import pallas as pl
from jax.experimental.pallas import tpu as pltpu
from jax.experimental.pallas import tpu_sc as plsc

HOP = 5
N = 10000
NPAD = 10240
H = 128
HH = 64
E = 320000
CH = 128
NCH = 160
D = 5
NPAIR = NCH // D
EPAD = 16 * NCH * CH
SLAB = NPAD // 16
DUMMY = 10016


def _lrelu(v):
    return jnp.where(v > 0, v, 0.2 * v)


def _agg_kernel(hap0, hap1, hpa0, hpa1, s_ap, t_ap, s_pa, t_pa, scal_ap, scal_pa,
                accap0, accap1, seg_ap, accpa0, accpa1, seg_pa,
                x1buf, h1buf, sidx, tidx, rows0, rows1, rows2, rows3, rows4,
                wbuf, accsh, segsh, *sems):
    sid = lax.axis_index("s")
    cid = lax.axis_index("c")
    zero16 = jnp.zeros((16,), jnp.float32)
    rows_bufs = (rows0, rows1, rows2, rows3, rows4)
    gsems = sems[0:D]
    ssems = sems[D:2 * D]
    isems = sems[2 * D:3 * D]

    def _run_type(h0, h1, s2d, t2d, scal, acc0_out, acc1_out, seg_out):
        pltpu.sync_copy(scal.at[0], x1buf)
        pltpu.sync_copy(scal.at[1], h1buf)
        tile_row0 = sid * NCH
        pltpu.sync_copy(s2d.at[pl.ds(tile_row0, NCH)], sidx)

        for p in range(2):
            hs = h0 if p == 0 else h1
            acc_out = acc0_out if p == 0 else acc1_out

            def _fire_idx(c, slot):
                pltpu.async_copy(t2d.at[tile_row0 + c], tidx.at[slot],
                                 isems[slot])

            def _wait_idx(c, slot):
                pltpu.make_async_copy(t2d.at[tile_row0 + c], tidx.at[slot],
                                      isems[slot]).wait()

            def _fire_gather(c, slot):
                pltpu.async_copy(hs.at[tidx.at[slot]], rows_bufs[slot],
                                 gsems[slot])

            def _wait_gather(c, slot):
                pltpu.make_async_copy(hs.at[tidx.at[slot]], rows_bufs[slot],
                                      gsems[slot]).wait()

            def _fire_scatter(c, slot):
                pltpu.async_copy(rows_bufs[slot], accsh.at[sidx.at[c]],
                                 ssems[slot], add=True)
                if p == 0:
                    pltpu.async_copy(wbuf.at[slot], segsh.at[sidx.at[c]],
                                     ssems[slot], add=True)

            def _wait_scatter(c, slot):
                pltpu.make_async_copy(rows_bufs[slot], accsh.at[sidx.at[c]],
                                      ssems[slot]).wait()
                if p == 0:
                    pltpu.make_async_copy(wbuf.at[slot], segsh.at[sidx.at[c]],
                                          ssems[slot]).wait()

            def _zrow(i, cc):
                for q in range(HH // 16):
                    rows0[i, pl.ds(q * 16, 16)] = zero16
                return cc
            lax.fori_loop(0, CH, _zrow, 0)
            for r in range(SLAB // CH):
                pltpu.sync_copy(rows0,
                                accsh.at[pl.ds(sid * SLAB + r * CH, CH)])
            if p == 0:
                for q in range(8):
                    wbuf[0, pl.ds(q * 16, 16)] = zero16
                for r in range(SLAB // 128):
                    pltpu.sync_copy(wbuf.at[0],
                                    segsh.at[pl.ds(sid * SLAB + r * 128, 128)])
            plsc.subcore_barrier()

            for c0 in range(4):
                _fire_idx(c0, c0 % D)
            _wait_idx(0, 0)
            _fire_gather(0, 0)
            _wait_idx(1, 1)
            _fire_gather(1, 1)

            def _round(g, carry):
                for db in range(D):
                    c = g * D + db
                    b = db

                    @pl.when(c >= 3)
                    def _():
                        _wait_scatter(c - 3, (db + 2) % D)

                    @pl.when(c + 4 < NCH)
                    def _():
                        _fire_idx(c + 4, (db + 4) % D)

                    @pl.when(c + 2 < NCH)
                    def _():
                        _wait_idx(c + 2, (db + 2) % D)
                        _fire_gather(c + 2, (db + 2) % D)

                    _wait_gather(c, b)
                    rows = rows_bufs[b]
                    def _wg(gg, cc):
                        col = gg * 16
                        sv = sidx[c, pl.ds(col, 16)]
                        tv = tidx[b, pl.ds(col, 16)]
                        xv = plsc.load_gather(x1buf, [sv])
                        hv = plsc.load_gather(h1buf, [tv])
                        wbuf[b, pl.ds(col, 16)] = (
                            jnp.exp(_lrelu(xv + hv)))
                        return cc
                    lax.fori_loop(0, 8, _wg, 0)
                    def _scale(gg, cc):
                        wv = wbuf[b, pl.ds(gg * 16, 16)]
                        for k in range(16):
                            e = gg * 16 + k
                            w = wv[k]
                            for q in range(HH // 16):
                                rows[e, pl.ds(q * 16, 16)] = (
                                    rows[e, pl.ds(q * 16, 16)] * w)
                        return cc
                    lax.fori_loop(0, 8, _scale, 0)
                    _fire_scatter(c, b)
                return carry

            lax.fori_loop(0, NPAIR, _round, 0)
            for c0 in (NCH - 3, NCH - 2, NCH - 1):
                _wait_scatter(c0, c0 % D)
            plsc.subcore_barrier()
            pltpu.sync_copy(accsh.at[pl.ds(sid * SLAB, SLAB)],
                            acc_out.at[pl.ds(sid * SLAB, SLAB)])
            if p == 0:
                pltpu.sync_copy(segsh.at[pl.ds(sid * SLAB, SLAB)],
                                seg_out.at[pl.ds(sid * SLAB, SLAB)])

    @pl.when(cid == 0)
    def _():
        _run_type(hap0, hap1, s_ap, t_ap, scal_ap, accap0, accap1, seg_ap)

    @pl.when(cid == 1)
    def _():
        _run_type(hpa0, hpa1, s_pa, t_pa, scal_pa, accpa0, accpa1, seg_pa)


_agg = functools.partial(
    pl.kernel,
    mesh=plsc.VectorSubcoreMesh(core_axis_name="c", subcore_axis_name="s"),
    compiler_params=pltpu.CompilerParams(
        needs_layout_passes=False, use_tc_tiling_on_sc=False),
    out_type=[
        jax.ShapeDtypeStruct((NPAD, HH), jnp.float32),
        jax.ShapeDtypeStruct((NPAD, HH), jnp.float32),
        jax.ShapeDtypeStruct((NPAD,), jnp.float32),
        jax.ShapeDtypeStruct((NPAD, HH), jnp.float32),
        jax.ShapeDtypeStruct((NPAD, HH), jnp.float32),
        jax.ShapeDtypeStruct((NPAD,), jnp.float32),
    ],
    scratch_types=[
        pltpu.VMEM((NPAD,), jnp.float32),
        pltpu.VMEM((NPAD,), jnp.float32),
        pltpu.VMEM((NCH, 128), jnp.int32),
        pltpu.VMEM((D, 128), jnp.int32),
        pltpu.VMEM((CH, HH), jnp.float32),
        pltpu.VMEM((CH, HH), jnp.float32),
        pltpu.VMEM((CH, HH), jnp.float32),
        pltpu.VMEM((CH, HH), jnp.float32),
        pltpu.VMEM((CH, HH), jnp.float32),
        pltpu.VMEM((D, 128), jnp.float32),
        pltpu.VMEM_SHARED((NPAD, HH), jnp.float32),
        pltpu.VMEM_SHARED((NPAD,), jnp.float32),
    ] + [pltpu.SemaphoreType.DMA] * 15,
)(_agg_kernel)


def _proj_body(h_ref, w_ref, b_ref, o_ref):
    o_ref[...] = h_ref[...] @ w_ref[...] + b_ref[...]


def _pad_edges(ei):
    s = jnp.concatenate([ei[0], jnp.full((EPAD - E,), DUMMY, jnp.int32)])
    t = jnp.concatenate([ei[1], jnp.zeros((EPAD - E,), jnp.int32)])
    return s.reshape(EPAD // 128, 128), t.reshape(EPAD // 128, 128)


def _pad_vec(v):
    return jnp.concatenate([v, jnp.zeros((NPAD - N,), jnp.float32)])


def kernel(x_author, x_paper, edge_index_ap, edge_index_pa, W1_author, b1_author, W1_paper, b1_paper, attn1, attn2, W_out, b_out):
    xa = jax.nn.relu(x_author @ W1_author + b1_author)
    xp = jax.nn.relu(x_paper @ W1_paper + b1_paper)
    s_ap, t_ap = _pad_edges(edge_index_ap)
    s_pa, t_pa = _pad_edges(edge_index_pa)

    A1_ap = attn1[:, 0, :].T
    A2_ap = attn2[:, 0, :].T
    A1_pa = attn1[:, 1, :].T
    A2_pa = attn2[:, 1, :].T
    x1_ap_all = xa @ A1_ap
    x1_pa_all = xp @ A1_pa
    w2_ap_all = jnp.exp(_lrelu(x1_ap_all + xa @ A2_ap))
    w2_pa_all = jnp.exp(_lrelu(x1_pa_all + xp @ A2_pa))

    ha, hp = xa, xp
    for i in range(HOP):
        h1_ap = _pad_vec(hp @ attn2[i, 0])
        h1_pa = _pad_vec(ha @ attn2[i, 1])
        scal_ap = jnp.stack([_pad_vec(x1_ap_all[:, i]), h1_ap])
        scal_pa = jnp.stack([_pad_vec(x1_pa_all[:, i]), h1_pa])
        acc_ap0, acc_ap1, seg_ap, acc_pa0, acc_pa1, seg_pa = _agg(
            hp[:, :HH], hp[:, HH:], ha[:, :HH], ha[:, HH:],
            s_ap, t_ap, s_pa, t_pa, scal_ap, scal_pa)
        acc_a = jnp.concatenate([acc_ap0[:N], acc_ap1[:N]], axis=1)
        acc_p = jnp.concatenate([acc_pa0[:N], acc_pa1[:N]], axis=1)
        w2a = w2_ap_all[:, i]
        w2p = w2_pa_all[:, i]
        hn_a = (acc_a + w2a[:, None] * xa) / (seg_ap[:N] + w2a)[:, None]
        hn_p = (acc_p + w2p[:, None] * xp) / (seg_pa[:N] + w2p)[:, None]
        ha = jax.nn.elu(hn_a)
        hp = jax.nn.elu(hn_p)

    BR = 400
    OUT = W_out.shape[1]
    return pl.pallas_call(
        _proj_body,
        grid=(N // BR,),
        in_specs=[
            pl.BlockSpec((BR, H), lambda i: (i, 0)),
            pl.BlockSpec((H, OUT), lambda i: (0, 0)),
            pl.BlockSpec((OUT,), lambda i: (0,)),
        ],
        out_specs=pl.BlockSpec((BR, OUT), lambda i: (i, 0)),
        out_shape=jax.ShapeDtypeStruct((N, OUT), jnp.float32),
    )(ha, W_out, b_out)
